# baseline (speedup 1.0000x reference)
import functools

import jax
import jax.numpy as jnp
import numpy as np
from jax.experimental import pallas as pl
from jax.experimental.pallas import tpu as pltpu

_N = 10000
_E = 160000
_F = 128
_H = 33
_C = 16
_HP = 48

_N_PAD = 10240
_ROW_BLK = 512


def _proj_body(x_ref, w_ref, s_ref, d_ref, xp_ref, as_ref, ad_ref, amax_ref):
    i = pl.program_id(0)
    xp = jnp.dot(x_ref[...], w_ref[...], preferred_element_type=jnp.float32)
    xp_ref[...] = xp
    a_s = jnp.dot(xp, s_ref[...], preferred_element_type=jnp.float32)
    a_d = jnp.dot(xp, d_ref[...], preferred_element_type=jnp.float32)
    as_ref[...] = a_s
    ad_ref[...] = a_d
    blk_max = jnp.max(a_s, axis=0, keepdims=True)

    @pl.when(i == 0)
    def _init():
        amax_ref[...] = blk_max

    @pl.when(i > 0)
    def _acc():
        amax_ref[...] = jnp.maximum(amax_ref[...], blk_max)


def _projection(x_pad, W, S, D):
    grid = _N_PAD // _ROW_BLK
    return pl.pallas_call(
        _proj_body,
        grid=(grid,),
        in_specs=[
            pl.BlockSpec((_ROW_BLK, _F), lambda i: (i, 0)),
            pl.BlockSpec((_F, _H * _C), lambda i: (0, 0)),
            pl.BlockSpec((_H * _C, _HP), lambda i: (0, 0)),
            pl.BlockSpec((_H * _C, _HP), lambda i: (0, 0)),
        ],
        out_specs=[
            pl.BlockSpec((_ROW_BLK, _H * _C), lambda i: (i, 0)),
            pl.BlockSpec((_ROW_BLK, _HP), lambda i: (i, 0)),
            pl.BlockSpec((_ROW_BLK, _HP), lambda i: (i, 0)),
            pl.BlockSpec((1, _HP), lambda i: (0, 0)),
        ],
        out_shape=[
            jax.ShapeDtypeStruct((_N_PAD, _H * _C), jnp.float32),
            jax.ShapeDtypeStruct((_N_PAD, _HP), jnp.float32),
            jax.ShapeDtypeStruct((_N_PAD, _HP), jnp.float32),
            jax.ShapeDtypeStruct((1, _HP), jnp.float32),
        ],
    )(x_pad, W, S, D)


def kernel(sensor_x, sensor_edge_index, W, att_src, att_dst, bias):
    H, C = _H, _C
    eye = jnp.eye(H, _HP, dtype=jnp.float32)
    S = (att_src.reshape(H, C, 1) * eye[:, None, :]).reshape(H * C, _HP)
    D = (att_dst.reshape(H, C, 1) * eye[:, None, :]).reshape(H * C, _HP)

    x_pad = jnp.zeros((_N_PAD, _F), jnp.float32).at[:_N].set(sensor_x)
    xp, a_s, a_d, amax = _projection(x_pad, W, S, D)

    loop = jnp.arange(_N, dtype=sensor_edge_index.dtype)
    ei = jnp.concatenate(
        [sensor_edge_index, jnp.stack([loop, loop], axis=0)], axis=1)
    src, dst = ei[0], ei[1]
    e = a_s[src, :H] + a_d[dst, :H]
    e = jax.nn.leaky_relu(e, negative_slope=0.2)
    mbound = jax.nn.leaky_relu(amax[0, :H] + a_d[dst, :H], negative_slope=0.2)
    ex = jnp.exp(e - mbound)
    denom = jax.ops.segment_sum(ex, dst, num_segments=_N)
    alpha = ex / (denom[dst] + 1e-16)
    msg = xp[src].reshape(-1, H, C) * alpha[:, :, None]
    out = jax.ops.segment_sum(msg, dst, num_segments=_N)
    out = out.mean(axis=1) + bias
    return jnp.tanh(out)


# speedup vs baseline: 161.5478x; 161.5478x over previous
import functools

import jax
import jax.numpy as jnp
import numpy as np
from jax import lax
from jax.experimental import pallas as pl
from jax.experimental.pallas import tpu as pltpu
from jax.experimental.pallas import tpu_sc as plsc

_N = 10000
_E = 160000
_F = 128
_H = 33
_C = 16
_HP = 48

_N_PAD = 10240
_ROW_BLK = 512


def _proj_body(x_ref, w_ref, s_ref, d_ref, xp_ref, as_ref, ad_ref, amax_ref):
    i = pl.program_id(0)
    xp = jnp.dot(x_ref[...], w_ref[...], preferred_element_type=jnp.float32)
    xp_ref[...] = xp
    a_s = jnp.dot(xp, s_ref[...], preferred_element_type=jnp.float32)
    a_d = jnp.dot(xp, d_ref[...], preferred_element_type=jnp.float32)
    as_ref[...] = a_s
    ad_ref[...] = a_d
    blk_max = jnp.max(a_s, axis=0, keepdims=True)

    @pl.when(i == 0)
    def _init():
        amax_ref[...] = blk_max

    @pl.when(i > 0)
    def _acc():
        amax_ref[...] = jnp.maximum(amax_ref[...], blk_max)


def _projection(x_pad, W, S, D):
    grid = _N_PAD // _ROW_BLK
    return pl.pallas_call(
        _proj_body,
        grid=(grid,),
        in_specs=[
            pl.BlockSpec((_ROW_BLK, _F), lambda i: (i, 0)),
            pl.BlockSpec((_F, _H * _C), lambda i: (0, 0)),
            pl.BlockSpec((_H * _C, _HP), lambda i: (0, 0)),
            pl.BlockSpec((_H * _C, _HP), lambda i: (0, 0)),
        ],
        out_specs=[
            pl.BlockSpec((_ROW_BLK, _H * _C), lambda i: (i, 0)),
            pl.BlockSpec((_ROW_BLK, _HP), lambda i: (i, 0)),
            pl.BlockSpec((_ROW_BLK, _HP), lambda i: (i, 0)),
            pl.BlockSpec((1, _HP), lambda i: (0, 0)),
        ],
        out_shape=[
            jax.ShapeDtypeStruct((_N_PAD, _H * _C), jnp.float32),
            jax.ShapeDtypeStruct((_N_PAD, _HP), jnp.float32),
            jax.ShapeDtypeStruct((_N_PAD, _HP), jnp.float32),
            jax.ShapeDtypeStruct((1, _HP), jnp.float32),
        ],
    )(x_pad, W, S, D)


_NC = 2
_NS = 16
_NW = _NC * _NS
_CHA = 128
_CPT_A = 42
_EPT = _CHA * _CPT_A
_E_PAD = _NW * _EPT
_ROWS_PT = _N_PAD // _NS


def _pass_a_body(src_hbm, dst_hbm, as_hbm, ad_hbm, amax_hbm,
                 ex_hbm, den_hbm,
                 src_v, dst_v, as_r, ad_r, ex_r, amax_v, zbuf, den_sp):
    cid = lax.axis_index("c")
    sid = lax.axis_index("s")
    wid = cid * _NS + sid

    def _z(i, _):
        for v in range(3):
            zbuf[i, pl.ds(v * 16, 16)] = jnp.zeros((16,), jnp.float32)
        return 0
    lax.fori_loop(0, 64, _z, 0)

    def _zs(k, _):
        pltpu.sync_copy(zbuf, den_sp.at[pl.ds(sid * _ROWS_PT + k * 64, 64)])
        return 0
    lax.fori_loop(0, _ROWS_PT // 64, _zs, 0)

    pltpu.sync_copy(amax_hbm, amax_v)
    amax_regs = [amax_v[pl.ds(v * 16, 16)] for v in range(3)]

    plsc.subcore_barrier()

    def _chunk(j, _):
        base = wid * _EPT + j * _CHA
        pltpu.sync_copy(src_hbm.at[pl.ds(base, _CHA)], src_v)
        pltpu.sync_copy(dst_hbm.at[pl.ds(base, _CHA)], dst_v)
        pltpu.sync_copy(as_hbm.at[src_v], as_r)
        pltpu.sync_copy(ad_hbm.at[dst_v], ad_r)

        def _edge(e, _):
            for v in range(3):
                sl = pl.ds(v * 16, 16)
                s = as_r[e, sl] + ad_r[e, sl]
                el = jnp.maximum(s, 0.2 * s)
                b = amax_regs[v] + ad_r[e, sl]
                bl = jnp.maximum(b, 0.2 * b)
                ex_r[e, sl] = jnp.exp(el - bl)
            return 0
        lax.fori_loop(0, _CHA, _edge, 0)

        pltpu.sync_copy(ex_r, ex_hbm.at[pl.ds(base, _CHA)])
        pltpu.sync_copy(ex_r, den_sp.at[dst_v], add=True)
        return 0
    lax.fori_loop(0, _CPT_A, _chunk, 0)

    plsc.subcore_barrier()
    pltpu.sync_copy(den_sp.at[pl.ds(sid * _ROWS_PT, _ROWS_PT)],
                    den_hbm.at[cid, pl.ds(sid * _ROWS_PT, _ROWS_PT)])


def _pass_a(src_pad, dst_pad, a_s, a_d, amax):
    mesh = plsc.VectorSubcoreMesh(core_axis_name="c", subcore_axis_name="s",
                                  num_cores=_NC, num_subcores=_NS)
    return pl.kernel(
        _pass_a_body,
        out_type=[
            jax.ShapeDtypeStruct((_E_PAD, _HP), jnp.float32),
            jax.ShapeDtypeStruct((_NC, _N_PAD, _HP), jnp.float32),
        ],
        mesh=mesh,
        scratch_types=[
            pltpu.VMEM((_CHA,), jnp.int32),
            pltpu.VMEM((_CHA,), jnp.int32),
            pltpu.VMEM((_CHA, _HP), jnp.float32),
            pltpu.VMEM((_CHA, _HP), jnp.float32),
            pltpu.VMEM((_CHA, _HP), jnp.float32),
            pltpu.VMEM((_HP,), jnp.float32),
            pltpu.VMEM((64, _HP), jnp.float32),
            pltpu.VMEM_SHARED((_N_PAD, _HP), jnp.float32),
        ],
        compiler_params=pltpu.CompilerParams(use_tc_tiling_on_sc=False),
        name="gat_edge_pass_a",
    )(src_pad, dst_pad, a_s, a_d, amax)


_CHB = 64
_CPT_B = _EPT // _CHB


def _pass_b_body(src_hbm, dst_hbm, ex_hbm, inv_hbm, xp_hbm,
                 out_hbm,
                 src_v, dst_v, ex_r, inv_r, al_r, xp_r, msg_r, zbuf, out_sp):
    cid = lax.axis_index("c")
    sid = lax.axis_index("s")
    wid = cid * _NS + sid

    def _z(i, _):
        zbuf[i, pl.ds(0, 16)] = jnp.zeros((16,), jnp.float32)
        zbuf[i, pl.ds(16, 16)] = jnp.zeros((16,), jnp.float32)
        msg_r[i, pl.ds(16, 16)] = jnp.zeros((16,), jnp.float32)
        return 0
    lax.fori_loop(0, 64, _z, 0)

    def _zs(k, _):
        pltpu.sync_copy(zbuf, out_sp.at[pl.ds(sid * _ROWS_PT + k * 64, 64)])
        return 0
    lax.fori_loop(0, _ROWS_PT // 64, _zs, 0)

    plsc.subcore_barrier()

    def _chunk(j, _):
        base = wid * _EPT + j * _CHB
        pltpu.sync_copy(src_hbm.at[pl.ds(base, _CHB)], src_v)
        pltpu.sync_copy(dst_hbm.at[pl.ds(base, _CHB)], dst_v)
        pltpu.sync_copy(ex_hbm.at[pl.ds(base, _CHB)], ex_r)
        pltpu.sync_copy(inv_hbm.at[dst_v], inv_r)
        pltpu.sync_copy(xp_hbm.at[src_v], xp_r)

        def _alpha(e, _):
            for v in range(3):
                sl = pl.ds(v * 16, 16)
                al_r[e, sl] = ex_r[e, sl] * inv_r[e, sl]
            return 0
        lax.fori_loop(0, _CHB, _alpha, 0)

        dnums = lax.GatherDimensionNumbers(
            offset_dims=(), collapsed_slice_dims=(0,), start_index_map=(0,))

        def _edge(e, _):
            acc = [jnp.zeros((16,), jnp.float32) for _ in range(4)]
            for v in range(3):
                av = al_r[e, pl.ds(v * 16, 16)]
                for h in range(v * 16, min((v + 1) * 16, _H)):
                    xv = xp_r[e, pl.ds(h * 16, 16)]
                    idx = jnp.full((16, 1), h % 16, jnp.int32)
                    ab = lax.gather(
                        av, idx, dnums, slice_sizes=(1,),
                        mode=lax.GatherScatterMode.PROMISE_IN_BOUNDS)
                    acc[h % 4] = acc[h % 4] + ab * xv
            msg_r[e, pl.ds(0, 16)] = (acc[0] + acc[1]) + (acc[2] + acc[3])
            return 0
        lax.fori_loop(0, _CHB, _edge, 0)

        pltpu.sync_copy(msg_r, out_sp.at[dst_v], add=True)
        return 0
    lax.fori_loop(0, _CPT_B, _chunk, 0)

    plsc.subcore_barrier()
    pltpu.sync_copy(out_sp.at[pl.ds(sid * _ROWS_PT, _ROWS_PT)],
                    out_hbm.at[cid, pl.ds(sid * _ROWS_PT, _ROWS_PT)])


def _pass_b(src_pad, dst_pad, ex_pad, inv_den, xp):
    mesh = plsc.VectorSubcoreMesh(core_axis_name="c", subcore_axis_name="s",
                                  num_cores=_NC, num_subcores=_NS)
    return pl.kernel(
        _pass_b_body,
        out_type=jax.ShapeDtypeStruct((_NC, _N_PAD, 32), jnp.float32),
        mesh=mesh,
        scratch_types=[
            pltpu.VMEM((_CHB,), jnp.int32),
            pltpu.VMEM((_CHB,), jnp.int32),
            pltpu.VMEM((_CHB, _HP), jnp.float32),
            pltpu.VMEM((_CHB, _HP), jnp.float32),
            pltpu.VMEM((_CHB, _HP), jnp.float32),
            pltpu.VMEM((_CHB, _H * _C), jnp.float32),
            pltpu.VMEM((_CHB, 32), jnp.float32),
            pltpu.VMEM((64, 32), jnp.float32),
            pltpu.VMEM_SHARED((_N_PAD, 32), jnp.float32),
        ],
        compiler_params=pltpu.CompilerParams(use_tc_tiling_on_sc=False),
        name="gat_edge_pass_b",
    )(src_pad, dst_pad, ex_pad, inv_den, xp)


def _inv_body(den_ref, inv_ref):
    inv_ref[...] = 1.0 / (den_ref[0] + den_ref[1] + 1e-16)


def _final_body(parts_ref, bias_ref, out_ref):
    s = (parts_ref[0] + parts_ref[1]) * (1.0 / _H)
    out_ref[...] = jnp.tanh(s + bias_ref[...])


def kernel(sensor_x, sensor_edge_index, W, att_src, att_dst, bias):
    H, C = _H, _C
    eye = jnp.eye(H, _HP, dtype=jnp.float32)
    S = (att_src.reshape(H, C, 1) * eye[:, None, :]).reshape(H * C, _HP)
    D = (att_dst.reshape(H, C, 1) * eye[:, None, :]).reshape(H * C, _HP)

    x_pad = jnp.zeros((_N_PAD, _F), jnp.float32).at[:_N].set(sensor_x)
    xp, a_s, a_d, amax = _projection(x_pad, W, S, D)

    loop = jnp.arange(_N, dtype=jnp.int32)
    trash = jnp.full((_E_PAD - _E - _N,), 10200, jnp.int32)
    src_pad = jnp.concatenate(
        [sensor_edge_index[0].astype(jnp.int32), loop, trash])
    dst_pad = jnp.concatenate(
        [sensor_edge_index[1].astype(jnp.int32), loop, trash])

    ex_pad, den_parts = _pass_a(src_pad, dst_pad, a_s, a_d,
                                amax.reshape(_HP))

    inv_den = pl.pallas_call(
        _inv_body,
        out_shape=jax.ShapeDtypeStruct((_N_PAD, _HP), jnp.float32),
    )(den_parts)

    out_parts = _pass_b(src_pad, dst_pad, ex_pad, inv_den, xp)

    bias_pad = jnp.concatenate([bias, jnp.zeros((32 - _C,), jnp.float32)])
    out = pl.pallas_call(
        _final_body,
        out_shape=jax.ShapeDtypeStruct((_N_PAD, 32), jnp.float32),
    )(out_parts, bias_pad.reshape(1, 32))
    return out[:_N, :_C]


# speedup vs baseline: 323.1147x; 2.0001x over previous
import functools

import jax
import jax.numpy as jnp
import numpy as np
from jax import lax
from jax.experimental import pallas as pl
from jax.experimental.pallas import tpu as pltpu
from jax.experimental.pallas import tpu_sc as plsc

_N = 10000
_E = 160000
_F = 128
_H = 33
_C = 16
_HP = 48

_N_PAD = 10240
_ROW_BLK = 512


def _proj_body(x_ref, w_ref, s_ref, d_ref, xp_ref, as_ref, ad_ref, amax_ref):
    i = pl.program_id(0)
    xp = jnp.dot(x_ref[...], w_ref[...], preferred_element_type=jnp.float32)
    xp_ref[...] = xp
    a_s = jnp.dot(xp, s_ref[...], preferred_element_type=jnp.float32)
    a_d = jnp.dot(xp, d_ref[...], preferred_element_type=jnp.float32)
    as_ref[...] = a_s
    ad_ref[...] = a_d
    blk_max = jnp.max(a_s, axis=0, keepdims=True)

    @pl.when(i == 0)
    def _init():
        amax_ref[...] = blk_max

    @pl.when(i > 0)
    def _acc():
        amax_ref[...] = jnp.maximum(amax_ref[...], blk_max)


def _projection(x_pad, W, S, D):
    grid = _N_PAD // _ROW_BLK
    return pl.pallas_call(
        _proj_body,
        grid=(grid,),
        in_specs=[
            pl.BlockSpec((_ROW_BLK, _F), lambda i: (i, 0)),
            pl.BlockSpec((_F, _H * _C), lambda i: (0, 0)),
            pl.BlockSpec((_H * _C, _HP), lambda i: (0, 0)),
            pl.BlockSpec((_H * _C, _HP), lambda i: (0, 0)),
        ],
        out_specs=[
            pl.BlockSpec((_ROW_BLK, _H * _C), lambda i: (i, 0)),
            pl.BlockSpec((_ROW_BLK, _HP), lambda i: (i, 0)),
            pl.BlockSpec((_ROW_BLK, _HP), lambda i: (i, 0)),
            pl.BlockSpec((1, _HP), lambda i: (0, 0)),
        ],
        out_shape=[
            jax.ShapeDtypeStruct((_N_PAD, _H * _C), jnp.float32),
            jax.ShapeDtypeStruct((_N_PAD, _HP), jnp.float32),
            jax.ShapeDtypeStruct((_N_PAD, _HP), jnp.float32),
            jax.ShapeDtypeStruct((1, _HP), jnp.float32),
        ],
    )(x_pad, W, S, D)


_NC = 2
_NS = 16
_NW = _NC * _NS
_CHA = 128
_CPT_A = 42
_EPT = _CHA * _CPT_A
_E_PAD = _NW * _EPT
_ROWS_PT = _N_PAD // _NS


def _pass_a_body(srcm_hbm, dstm_hbm, as_hbm, ad_hbm, amax_hbm,
                 ex_hbm, den_hbm,
                 src_all, dst_all,
                 as_r0, as_r1, ad_r0, ad_r1, ex_r0, ex_r1,
                 amax_v, zbuf, den_sp, sems):
    cid = lax.axis_index("c")
    sid = lax.axis_index("s")
    wid = cid * _NS + sid
    as_b, ad_b, ex_b = [as_r0, as_r1], [ad_r0, ad_r1], [ex_r0, ex_r1]

    def _z(i, _):
        for v in range(3):
            zbuf[i, pl.ds(v * 16, 16)] = jnp.zeros((16,), jnp.float32)
        return 0
    lax.fori_loop(0, 64, _z, 0)

    def _zs(k, _):
        pltpu.sync_copy(zbuf, den_sp.at[pl.ds(sid * _ROWS_PT + k * 64, 64)])
        return 0
    lax.fori_loop(0, _ROWS_PT // 64, _zs, 0)

    pltpu.sync_copy(amax_hbm, amax_v)
    amax_regs = [amax_v[pl.ds(v * 16, 16)] for v in range(3)]
    pltpu.sync_copy(srcm_hbm.at[pl.ds(wid * _CPT_A, _CPT_A)], src_all)
    pltpu.sync_copy(dstm_hbm.at[pl.ds(wid * _CPT_A, _CPT_A)], dst_all)

    plsc.subcore_barrier()

    def _issue(j, b):
        pltpu.async_copy(as_hbm.at[src_all.at[j]], as_b[b], sems.at[b])
        pltpu.async_copy(ad_hbm.at[dst_all.at[j]], ad_b[b], sems.at[2 + b])

    def _wait_gather(j, b):
        pltpu.make_async_copy(as_hbm.at[src_all.at[j]], as_b[b],
                              sems.at[b]).wait()
        pltpu.make_async_copy(ad_hbm.at[dst_all.at[j]], ad_b[b],
                              sems.at[2 + b]).wait()

    def _wait_outputs(j, b):
        base = wid * _EPT + j * _CHA
        pltpu.make_async_copy(ex_b[b], ex_hbm.at[pl.ds(base, _CHA)],
                              sems.at[4 + b]).wait()
        pltpu.make_async_copy(ex_b[b], den_sp.at[dst_all.at[j]],
                              sems.at[6 + b]).wait()

    _issue(0, 0)

    @pl.loop(0, _CPT_A, step=2)
    def _outer(jj):
        for b in range(2):
            j = jj + b

            @pl.when(j + 1 < _CPT_A)
            def _():
                _issue(j + 1, 1 - b)

            _wait_gather(j, b)

            @pl.when(j >= 2)
            def _():
                _wait_outputs(j - 2, b)

            as_r, ad_r, ex_r = as_b[b], ad_b[b], ex_b[b]

            def _edge(e, _):
                for v in range(3):
                    sl = pl.ds(v * 16, 16)
                    s = as_r[e, sl] + ad_r[e, sl]
                    el = jnp.maximum(s, 0.2 * s)
                    bb = amax_regs[v] + ad_r[e, sl]
                    bl = jnp.maximum(bb, 0.2 * bb)
                    ex_r[e, sl] = jnp.exp(el - bl)
                return 0
            lax.fori_loop(0, _CHA, _edge, 0)

            base = wid * _EPT + j * _CHA
            pltpu.async_copy(ex_r, ex_hbm.at[pl.ds(base, _CHA)],
                             sems.at[4 + b])
            pltpu.async_copy(ex_r, den_sp.at[dst_all.at[j]],
                             sems.at[6 + b], add=True)

    for b in range(2):
        _wait_outputs(_CPT_A - 2 + b, b)

    plsc.subcore_barrier()
    pltpu.sync_copy(den_sp.at[pl.ds(sid * _ROWS_PT, _ROWS_PT)],
                    den_hbm.at[cid, pl.ds(sid * _ROWS_PT, _ROWS_PT)])


def _pass_a(srcm, dstm, a_s, a_d, amax):
    mesh = plsc.VectorSubcoreMesh(core_axis_name="c", subcore_axis_name="s",
                                  num_cores=_NC, num_subcores=_NS)
    return pl.kernel(
        _pass_a_body,
        out_type=[
            jax.ShapeDtypeStruct((_E_PAD, _HP), jnp.float32),
            jax.ShapeDtypeStruct((_NC, _N_PAD, _HP), jnp.float32),
        ],
        mesh=mesh,
        scratch_types=[
            pltpu.VMEM((_CPT_A, _CHA), jnp.int32),
            pltpu.VMEM((_CPT_A, _CHA), jnp.int32),
            pltpu.VMEM((_CHA, _HP), jnp.float32),
            pltpu.VMEM((_CHA, _HP), jnp.float32),
            pltpu.VMEM((_CHA, _HP), jnp.float32),
            pltpu.VMEM((_CHA, _HP), jnp.float32),
            pltpu.VMEM((_CHA, _HP), jnp.float32),
            pltpu.VMEM((_CHA, _HP), jnp.float32),
            pltpu.VMEM((_HP,), jnp.float32),
            pltpu.VMEM((64, _HP), jnp.float32),
            pltpu.VMEM_SHARED((_N_PAD, _HP), jnp.float32),
            pltpu.SemaphoreType.DMA((8,)),
        ],
        compiler_params=pltpu.CompilerParams(use_tc_tiling_on_sc=False),
        name="gat_edge_pass_a",
    )(srcm, dstm, a_s, a_d, amax)


_CHB = 64
_CPT_B = _EPT // _CHB


def _pass_b_body(srcm_hbm, dstm_hbm, ex_hbm, inv_hbm, xp_hbm,
                 out_hbm,
                 src_all, dst_all,
                 ex_r0, ex_r1, inv_r0, inv_r1, xp_r0, xp_r1,
                 al_r, msg_r0, msg_r1, zbuf, out_sp, sems):
    cid = lax.axis_index("c")
    sid = lax.axis_index("s")
    wid = cid * _NS + sid
    ex_b, inv_b, xp_b = [ex_r0, ex_r1], [inv_r0, inv_r1], [xp_r0, xp_r1]
    msg_b = [msg_r0, msg_r1]

    def _z(i, _):
        zbuf[i, pl.ds(0, 16)] = jnp.zeros((16,), jnp.float32)
        zbuf[i, pl.ds(16, 16)] = jnp.zeros((16,), jnp.float32)
        msg_r0[i, pl.ds(16, 16)] = jnp.zeros((16,), jnp.float32)
        msg_r1[i, pl.ds(16, 16)] = jnp.zeros((16,), jnp.float32)
        return 0
    lax.fori_loop(0, 64, _z, 0)

    def _zs(k, _):
        pltpu.sync_copy(zbuf, out_sp.at[pl.ds(sid * _ROWS_PT + k * 64, 64)])
        return 0
    lax.fori_loop(0, _ROWS_PT // 64, _zs, 0)

    pltpu.sync_copy(srcm_hbm.at[pl.ds(wid * _CPT_B, _CPT_B)], src_all)
    pltpu.sync_copy(dstm_hbm.at[pl.ds(wid * _CPT_B, _CPT_B)], dst_all)

    plsc.subcore_barrier()

    def _issue(j, b):
        base = wid * _EPT + j * _CHB
        pltpu.async_copy(ex_hbm.at[pl.ds(base, _CHB)], ex_b[b], sems.at[b])
        pltpu.async_copy(inv_hbm.at[dst_all.at[j]], inv_b[b], sems.at[2 + b])
        pltpu.async_copy(xp_hbm.at[src_all.at[j]], xp_b[b], sems.at[4 + b])

    def _wait_gather(j, b):
        base = wid * _EPT + j * _CHB
        pltpu.make_async_copy(ex_hbm.at[pl.ds(base, _CHB)], ex_b[b],
                              sems.at[b]).wait()
        pltpu.make_async_copy(inv_hbm.at[dst_all.at[j]], inv_b[b],
                              sems.at[2 + b]).wait()
        pltpu.make_async_copy(xp_hbm.at[src_all.at[j]], xp_b[b],
                              sems.at[4 + b]).wait()

    def _wait_scatter(j, b):
        pltpu.make_async_copy(msg_b[b], out_sp.at[dst_all.at[j]],
                              sems.at[6 + b]).wait()

    _issue(0, 0)

    dnums = lax.GatherDimensionNumbers(
        offset_dims=(), collapsed_slice_dims=(0,), start_index_map=(0,))

    @pl.loop(0, _CPT_B, step=2)
    def _outer(jj):
        for b in range(2):
            j = jj + b

            @pl.when(j + 1 < _CPT_B)
            def _():
                _issue(j + 1, 1 - b)

            _wait_gather(j, b)

            ex_r, inv_r, xp_r, msg_r = ex_b[b], inv_b[b], xp_b[b], msg_b[b]

            def _alpha(e, _):
                for v in range(3):
                    sl = pl.ds(v * 16, 16)
                    al_r[e, sl] = ex_r[e, sl] * inv_r[e, sl]
                return 0
            lax.fori_loop(0, _CHB, _alpha, 0)

            @pl.when(j >= 2)
            def _():
                _wait_scatter(j - 2, b)

            def _edge(e, _):
                acc = [jnp.zeros((16,), jnp.float32) for _ in range(4)]
                for v in range(3):
                    av = al_r[e, pl.ds(v * 16, 16)]
                    for h in range(v * 16, min((v + 1) * 16, _H)):
                        xv = xp_r[e, pl.ds(h * 16, 16)]
                        idx = jnp.full((16, 1), h % 16, jnp.int32)
                        ab = lax.gather(
                            av, idx, dnums, slice_sizes=(1,),
                            mode=lax.GatherScatterMode.PROMISE_IN_BOUNDS)
                        acc[h % 4] = acc[h % 4] + ab * xv
                msg_r[e, pl.ds(0, 16)] = (acc[0] + acc[1]) + (acc[2] + acc[3])
                return 0
            lax.fori_loop(0, _CHB, _edge, 0)

            pltpu.async_copy(msg_r, out_sp.at[dst_all.at[j]],
                             sems.at[6 + b], add=True)

    for b in range(2):
        _wait_scatter(_CPT_B - 2 + b, b)

    plsc.subcore_barrier()
    pltpu.sync_copy(out_sp.at[pl.ds(sid * _ROWS_PT, _ROWS_PT)],
                    out_hbm.at[cid, pl.ds(sid * _ROWS_PT, _ROWS_PT)])


def _pass_b(srcm, dstm, ex_pad, inv_den, xp):
    mesh = plsc.VectorSubcoreMesh(core_axis_name="c", subcore_axis_name="s",
                                  num_cores=_NC, num_subcores=_NS)
    return pl.kernel(
        _pass_b_body,
        out_type=jax.ShapeDtypeStruct((_NC, _N_PAD, 32), jnp.float32),
        mesh=mesh,
        scratch_types=[
            pltpu.VMEM((_CPT_B, _CHB), jnp.int32),
            pltpu.VMEM((_CPT_B, _CHB), jnp.int32),
            pltpu.VMEM((_CHB, _HP), jnp.float32),
            pltpu.VMEM((_CHB, _HP), jnp.float32),
            pltpu.VMEM((_CHB, _HP), jnp.float32),
            pltpu.VMEM((_CHB, _HP), jnp.float32),
            pltpu.VMEM((_CHB, _H * _C), jnp.float32),
            pltpu.VMEM((_CHB, _H * _C), jnp.float32),
            pltpu.VMEM((_CHB, _HP), jnp.float32),
            pltpu.VMEM((_CHB, 32), jnp.float32),
            pltpu.VMEM((_CHB, 32), jnp.float32),
            pltpu.VMEM((64, 32), jnp.float32),
            pltpu.VMEM_SHARED((_N_PAD, 32), jnp.float32),
            pltpu.SemaphoreType.DMA((8,)),
        ],
        compiler_params=pltpu.CompilerParams(use_tc_tiling_on_sc=False),
        name="gat_edge_pass_b",
    )(srcm, dstm, ex_pad, inv_den, xp)


def _inv_body(den_ref, inv_ref):
    inv_ref[...] = 1.0 / (den_ref[0] + den_ref[1] + 1e-16)


def _final_body(parts_ref, bias_ref, out_ref):
    s = (parts_ref[0] + parts_ref[1]) * (1.0 / _H)
    out_ref[...] = jnp.tanh(s + bias_ref[...])


def kernel(sensor_x, sensor_edge_index, W, att_src, att_dst, bias):
    H, C = _H, _C
    eye = jnp.eye(H, _HP, dtype=jnp.float32)
    S = (att_src.reshape(H, C, 1) * eye[:, None, :]).reshape(H * C, _HP)
    D = (att_dst.reshape(H, C, 1) * eye[:, None, :]).reshape(H * C, _HP)

    x_pad = jnp.zeros((_N_PAD, _F), jnp.float32).at[:_N].set(sensor_x)
    xp, a_s, a_d, amax = _projection(x_pad, W, S, D)

    loop = jnp.arange(_N, dtype=jnp.int32)
    trash = jnp.full((_E_PAD - _E - _N,), 10200, jnp.int32)
    src_pad = jnp.concatenate(
        [sensor_edge_index[0].astype(jnp.int32), loop, trash])
    dst_pad = jnp.concatenate(
        [sensor_edge_index[1].astype(jnp.int32), loop, trash])

    srcm_a = src_pad.reshape(_E_PAD // _CHA, _CHA)
    dstm_a = dst_pad.reshape(_E_PAD // _CHA, _CHA)
    srcm_b = src_pad.reshape(_E_PAD // _CHB, _CHB)
    dstm_b = dst_pad.reshape(_E_PAD // _CHB, _CHB)

    ex_pad, den_parts = _pass_a(srcm_a, dstm_a, a_s, a_d,
                                amax.reshape(_HP))

    inv_den = pl.pallas_call(
        _inv_body,
        out_shape=jax.ShapeDtypeStruct((_N_PAD, _HP), jnp.float32),
    )(den_parts)

    out_parts = _pass_b(srcm_b, dstm_b, ex_pad, inv_den, xp)

    bias_pad = jnp.concatenate([bias, jnp.zeros((32 - _C,), jnp.float32)])
    out = pl.pallas_call(
        _final_body,
        out_shape=jax.ShapeDtypeStruct((_N_PAD, 32), jnp.float32),
    )(out_parts, bias_pad.reshape(1, 32))
    return out[:_N, :_C]


# speedup vs baseline: 448.6198x; 1.3884x over previous
import functools

import jax
import jax.numpy as jnp
import numpy as np
from jax import lax
from jax.experimental import pallas as pl
from jax.experimental.pallas import tpu as pltpu
from jax.experimental.pallas import tpu_sc as plsc

_N = 10000
_E = 160000
_F = 128
_H = 33
_C = 16
_HP = 48

_N_PAD = 10240
_ROW_BLK = 512


def _proj_body(x_ref, w_ref, s_ref, d_ref, xp_ref, as_ref, ad_ref, amax_ref):
    i = pl.program_id(0)
    xp = jnp.dot(x_ref[...], w_ref[...], preferred_element_type=jnp.float32)
    xp_ref[...] = xp
    a_s = jnp.dot(xp, s_ref[...], preferred_element_type=jnp.float32)
    a_d = jnp.dot(xp, d_ref[...], preferred_element_type=jnp.float32)
    as_ref[...] = a_s
    ad_ref[...] = a_d
    blk_max = jnp.max(a_s, axis=0, keepdims=True)

    @pl.when(i == 0)
    def _init():
        amax_ref[...] = blk_max

    @pl.when(i > 0)
    def _acc():
        amax_ref[...] = jnp.maximum(amax_ref[...], blk_max)


def _projection(x_pad, W, S, D):
    grid = _N_PAD // _ROW_BLK
    return pl.pallas_call(
        _proj_body,
        grid=(grid,),
        in_specs=[
            pl.BlockSpec((_ROW_BLK, _F), lambda i: (i, 0)),
            pl.BlockSpec((_F, _H * _C), lambda i: (0, 0)),
            pl.BlockSpec((_H * _C, _HP), lambda i: (0, 0)),
            pl.BlockSpec((_H * _C, _HP), lambda i: (0, 0)),
        ],
        out_specs=[
            pl.BlockSpec((_ROW_BLK, _H * _C), lambda i: (i, 0)),
            pl.BlockSpec((_ROW_BLK, _HP), lambda i: (i, 0)),
            pl.BlockSpec((_ROW_BLK, _HP), lambda i: (i, 0)),
            pl.BlockSpec((1, _HP), lambda i: (0, 0)),
        ],
        out_shape=[
            jax.ShapeDtypeStruct((_N_PAD, _H * _C), jnp.float32),
            jax.ShapeDtypeStruct((_N_PAD, _HP), jnp.float32),
            jax.ShapeDtypeStruct((_N_PAD, _HP), jnp.float32),
            jax.ShapeDtypeStruct((1, _HP), jnp.float32),
        ],
    )(x_pad, W, S, D)


_NC = 2
_NS = 16
_NW = _NC * _NS
_CHA = 128
_CPT_A = 42
_EPT = _CHA * _CPT_A
_E_PAD = _NW * _EPT
_ROWS_PT = _N_PAD // _NS


def _pass_a_body(srcm_hbm, dstm_hbm, as_hbm, ad_hbm, amax_hbm,
                 ex_hbm, den_hbm,
                 src_all, dst_all,
                 as_r0, as_r1, ad_r0, ad_r1, ex_r0, ex_r1,
                 amax_v, zbuf, den_sp, sems):
    cid = lax.axis_index("c")
    sid = lax.axis_index("s")
    wid = cid * _NS + sid
    as_b, ad_b, ex_b = [as_r0, as_r1], [ad_r0, ad_r1], [ex_r0, ex_r1]

    def _z(i, _):
        for v in range(3):
            zbuf[i, pl.ds(v * 16, 16)] = jnp.zeros((16,), jnp.float32)
        return 0
    lax.fori_loop(0, 64, _z, 0)

    def _zs(k, _):
        pltpu.sync_copy(zbuf, den_sp.at[pl.ds(sid * _ROWS_PT + k * 64, 64)])
        return 0
    lax.fori_loop(0, _ROWS_PT // 64, _zs, 0)

    pltpu.sync_copy(amax_hbm, amax_v)
    amax_regs = [amax_v[pl.ds(v * 16, 16)] for v in range(3)]
    pltpu.sync_copy(srcm_hbm.at[pl.ds(wid * _CPT_A, _CPT_A)], src_all)
    pltpu.sync_copy(dstm_hbm.at[pl.ds(wid * _CPT_A, _CPT_A)], dst_all)

    plsc.subcore_barrier()

    def _issue(j, b):
        pltpu.async_copy(as_hbm.at[src_all.at[j]], as_b[b], sems.at[b])
        pltpu.async_copy(ad_hbm.at[dst_all.at[j]], ad_b[b], sems.at[2 + b])

    def _wait_gather(j, b):
        pltpu.make_async_copy(as_hbm.at[src_all.at[j]], as_b[b],
                              sems.at[b]).wait()
        pltpu.make_async_copy(ad_hbm.at[dst_all.at[j]], ad_b[b],
                              sems.at[2 + b]).wait()

    def _wait_outputs(j, b):
        base = wid * _EPT + j * _CHA
        pltpu.make_async_copy(ex_b[b], ex_hbm.at[pl.ds(base, _CHA)],
                              sems.at[4 + b]).wait()
        pltpu.make_async_copy(ex_b[b], den_sp.at[dst_all.at[j]],
                              sems.at[6 + b]).wait()

    _issue(0, 0)

    @pl.loop(0, _CPT_A, step=2)
    def _outer(jj):
        for b in range(2):
            j = jj + b

            @pl.when(j + 1 < _CPT_A)
            def _():
                _issue(j + 1, 1 - b)

            _wait_gather(j, b)

            @pl.when(j >= 2)
            def _():
                _wait_outputs(j - 2, b)

            as_r, ad_r, ex_r = as_b[b], ad_b[b], ex_b[b]

            @plsc.parallel_loop(0, _CHA, unroll=2)
            def _edge(e):
                for v in range(3):
                    sl = pl.ds(v * 16, 16)
                    s = as_r[e, sl] + ad_r[e, sl]
                    el = jnp.maximum(s, 0.2 * s)
                    bb = amax_regs[v] + ad_r[e, sl]
                    bl = jnp.maximum(bb, 0.2 * bb)
                    ex_r[e, sl] = jnp.exp(el - bl)

            base = wid * _EPT + j * _CHA
            pltpu.async_copy(ex_r, ex_hbm.at[pl.ds(base, _CHA)],
                             sems.at[4 + b])
            pltpu.async_copy(ex_r, den_sp.at[dst_all.at[j]],
                             sems.at[6 + b], add=True)

    for b in range(2):
        _wait_outputs(_CPT_A - 2 + b, b)

    plsc.subcore_barrier()
    pltpu.sync_copy(den_sp.at[pl.ds(sid * _ROWS_PT, _ROWS_PT)],
                    den_hbm.at[cid, pl.ds(sid * _ROWS_PT, _ROWS_PT)])


def _pass_a(srcm, dstm, a_s, a_d, amax):
    mesh = plsc.VectorSubcoreMesh(core_axis_name="c", subcore_axis_name="s",
                                  num_cores=_NC, num_subcores=_NS)
    return pl.kernel(
        _pass_a_body,
        out_type=[
            jax.ShapeDtypeStruct((_E_PAD, _HP), jnp.float32),
            jax.ShapeDtypeStruct((_NC, _N_PAD, _HP), jnp.float32),
        ],
        mesh=mesh,
        scratch_types=[
            pltpu.VMEM((_CPT_A, _CHA), jnp.int32),
            pltpu.VMEM((_CPT_A, _CHA), jnp.int32),
            pltpu.VMEM((_CHA, _HP), jnp.float32),
            pltpu.VMEM((_CHA, _HP), jnp.float32),
            pltpu.VMEM((_CHA, _HP), jnp.float32),
            pltpu.VMEM((_CHA, _HP), jnp.float32),
            pltpu.VMEM((_CHA, _HP), jnp.float32),
            pltpu.VMEM((_CHA, _HP), jnp.float32),
            pltpu.VMEM((_HP,), jnp.float32),
            pltpu.VMEM((64, _HP), jnp.float32),
            pltpu.VMEM_SHARED((_N_PAD, _HP), jnp.float32),
            pltpu.SemaphoreType.DMA((8,)),
        ],
        compiler_params=pltpu.CompilerParams(use_tc_tiling_on_sc=False),
        name="gat_edge_pass_a",
    )(srcm, dstm, a_s, a_d, amax)


_CHB = 64
_CPT_B = _EPT // _CHB


def _pass_b_body(srcm_hbm, dstm_hbm, ex_hbm, inv_hbm, xp_hbm,
                 out_hbm,
                 src_all, dst_all,
                 ex_r0, ex_r1, inv_r0, inv_r1, xp_r0, xp_r1,
                 al_r, msg_r0, msg_r1, zbuf, out_sp, sems):
    cid = lax.axis_index("c")
    sid = lax.axis_index("s")
    wid = cid * _NS + sid
    ex_b, inv_b, xp_b = [ex_r0, ex_r1], [inv_r0, inv_r1], [xp_r0, xp_r1]
    msg_b = [msg_r0, msg_r1]

    def _z(i, _):
        zbuf[i, pl.ds(0, 16)] = jnp.zeros((16,), jnp.float32)
        zbuf[i, pl.ds(16, 16)] = jnp.zeros((16,), jnp.float32)
        msg_r0[i, pl.ds(16, 16)] = jnp.zeros((16,), jnp.float32)
        msg_r1[i, pl.ds(16, 16)] = jnp.zeros((16,), jnp.float32)
        return 0
    lax.fori_loop(0, 64, _z, 0)

    def _zs(k, _):
        pltpu.sync_copy(zbuf, out_sp.at[pl.ds(sid * _ROWS_PT + k * 64, 64)])
        return 0
    lax.fori_loop(0, _ROWS_PT // 64, _zs, 0)

    pltpu.sync_copy(srcm_hbm.at[pl.ds(wid * _CPT_B, _CPT_B)], src_all)
    pltpu.sync_copy(dstm_hbm.at[pl.ds(wid * _CPT_B, _CPT_B)], dst_all)

    plsc.subcore_barrier()

    def _issue(j, b):
        base = wid * _EPT + j * _CHB
        pltpu.async_copy(ex_hbm.at[pl.ds(base, _CHB)], ex_b[b], sems.at[b])
        pltpu.async_copy(inv_hbm.at[dst_all.at[j]], inv_b[b], sems.at[2 + b])
        pltpu.async_copy(xp_hbm.at[src_all.at[j]], xp_b[b], sems.at[4 + b])

    def _wait_gather(j, b):
        base = wid * _EPT + j * _CHB
        pltpu.make_async_copy(ex_hbm.at[pl.ds(base, _CHB)], ex_b[b],
                              sems.at[b]).wait()
        pltpu.make_async_copy(inv_hbm.at[dst_all.at[j]], inv_b[b],
                              sems.at[2 + b]).wait()
        pltpu.make_async_copy(xp_hbm.at[src_all.at[j]], xp_b[b],
                              sems.at[4 + b]).wait()

    def _wait_scatter(j, b):
        pltpu.make_async_copy(msg_b[b], out_sp.at[dst_all.at[j]],
                              sems.at[6 + b]).wait()

    _issue(0, 0)

    dnums = lax.GatherDimensionNumbers(
        offset_dims=(), collapsed_slice_dims=(0,), start_index_map=(0,))

    @pl.loop(0, _CPT_B, step=2)
    def _outer(jj):
        for b in range(2):
            j = jj + b

            @pl.when(j + 1 < _CPT_B)
            def _():
                _issue(j + 1, 1 - b)

            _wait_gather(j, b)

            ex_r, inv_r, xp_r, msg_r = ex_b[b], inv_b[b], xp_b[b], msg_b[b]

            @plsc.parallel_loop(0, _CHB, unroll=2)
            def _alpha(e):
                for v in range(3):
                    sl = pl.ds(v * 16, 16)
                    al_r[e, sl] = ex_r[e, sl] * inv_r[e, sl]

            @pl.when(j >= 2)
            def _():
                _wait_scatter(j - 2, b)

            @plsc.parallel_loop(0, _CHB, unroll=2)
            def _edge(e):
                acc = [jnp.zeros((16,), jnp.float32) for _ in range(4)]
                for v in range(3):
                    av = al_r[e, pl.ds(v * 16, 16)]
                    for h in range(v * 16, min((v + 1) * 16, _H)):
                        xv = xp_r[e, pl.ds(h * 16, 16)]
                        idx = jnp.full((16, 1), h % 16, jnp.int32)
                        ab = lax.gather(
                            av, idx, dnums, slice_sizes=(1,),
                            mode=lax.GatherScatterMode.PROMISE_IN_BOUNDS)
                        acc[h % 4] = acc[h % 4] + ab * xv
                msg_r[e, pl.ds(0, 16)] = (acc[0] + acc[1]) + (acc[2] + acc[3])

            pltpu.async_copy(msg_r, out_sp.at[dst_all.at[j]],
                             sems.at[6 + b], add=True)

    for b in range(2):
        _wait_scatter(_CPT_B - 2 + b, b)

    plsc.subcore_barrier()
    pltpu.sync_copy(out_sp.at[pl.ds(sid * _ROWS_PT, _ROWS_PT)],
                    out_hbm.at[cid, pl.ds(sid * _ROWS_PT, _ROWS_PT)])


def _pass_b(srcm, dstm, ex_pad, inv_den, xp):
    mesh = plsc.VectorSubcoreMesh(core_axis_name="c", subcore_axis_name="s",
                                  num_cores=_NC, num_subcores=_NS)
    return pl.kernel(
        _pass_b_body,
        out_type=jax.ShapeDtypeStruct((_NC, _N_PAD, 32), jnp.float32),
        mesh=mesh,
        scratch_types=[
            pltpu.VMEM((_CPT_B, _CHB), jnp.int32),
            pltpu.VMEM((_CPT_B, _CHB), jnp.int32),
            pltpu.VMEM((_CHB, _HP), jnp.float32),
            pltpu.VMEM((_CHB, _HP), jnp.float32),
            pltpu.VMEM((_CHB, _HP), jnp.float32),
            pltpu.VMEM((_CHB, _HP), jnp.float32),
            pltpu.VMEM((_CHB, _H * _C), jnp.float32),
            pltpu.VMEM((_CHB, _H * _C), jnp.float32),
            pltpu.VMEM((_CHB, _HP), jnp.float32),
            pltpu.VMEM((_CHB, 32), jnp.float32),
            pltpu.VMEM((_CHB, 32), jnp.float32),
            pltpu.VMEM((64, 32), jnp.float32),
            pltpu.VMEM_SHARED((_N_PAD, 32), jnp.float32),
            pltpu.SemaphoreType.DMA((8,)),
        ],
        compiler_params=pltpu.CompilerParams(use_tc_tiling_on_sc=False),
        name="gat_edge_pass_b",
    )(srcm, dstm, ex_pad, inv_den, xp)


def _inv_body(den_ref, inv_ref):
    inv_ref[...] = 1.0 / (den_ref[0] + den_ref[1] + 1e-16)


def _final_body(parts_ref, bias_ref, out_ref):
    s = (parts_ref[0] + parts_ref[1]) * (1.0 / _H)
    out_ref[...] = jnp.tanh(s + bias_ref[...])


def kernel(sensor_x, sensor_edge_index, W, att_src, att_dst, bias):
    H, C = _H, _C
    eye = jnp.eye(H, _HP, dtype=jnp.float32)
    S = (att_src.reshape(H, C, 1) * eye[:, None, :]).reshape(H * C, _HP)
    D = (att_dst.reshape(H, C, 1) * eye[:, None, :]).reshape(H * C, _HP)

    x_pad = jnp.zeros((_N_PAD, _F), jnp.float32).at[:_N].set(sensor_x)
    xp, a_s, a_d, amax = _projection(x_pad, W, S, D)

    loop = jnp.arange(_N, dtype=jnp.int32)
    trash = 10000 + (jnp.arange(_E_PAD - _E - _N, dtype=jnp.int32) % 240)
    src_pad = jnp.concatenate(
        [sensor_edge_index[0].astype(jnp.int32), loop, trash])
    dst_pad = jnp.concatenate(
        [sensor_edge_index[1].astype(jnp.int32), loop, trash])

    srcm_a = src_pad.reshape(_E_PAD // _CHA, _CHA)
    dstm_a = dst_pad.reshape(_E_PAD // _CHA, _CHA)
    srcm_b = src_pad.reshape(_E_PAD // _CHB, _CHB)
    dstm_b = dst_pad.reshape(_E_PAD // _CHB, _CHB)

    ex_pad, den_parts = _pass_a(srcm_a, dstm_a, a_s, a_d,
                                amax.reshape(_HP))

    inv_den = pl.pallas_call(
        _inv_body,
        out_shape=jax.ShapeDtypeStruct((_N_PAD, _HP), jnp.float32),
    )(den_parts)

    out_parts = _pass_b(srcm_b, dstm_b, ex_pad, inv_den, xp)

    bias_pad = jnp.concatenate([bias, jnp.zeros((32 - _C,), jnp.float32)])
    out = pl.pallas_call(
        _final_body,
        out_shape=jax.ShapeDtypeStruct((_N_PAD, 32), jnp.float32),
    )(out_parts, bias_pad.reshape(1, 32))
    return out[:_N, :_C]
